# baseline (speedup 1.0000x reference)
import functools

import jax
import jax.numpy as jnp
from jax import lax
from jax.experimental import pallas as pl
from jax.experimental.pallas import tpu as pltpu
from jax.experimental.pallas import tpu_sc as plsc

_NORM_START = 5000
_NQ = 8
_TPAD = 512


def _leaky(x):
    return jnp.where(x >= 0, x, 0.01 * x)


def _argmin_onehot(d, k):
    dmin = jnp.min(d, axis=1, keepdims=True)
    iota = lax.broadcasted_iota(jnp.int32, d.shape, 1)
    idx = jnp.min(jnp.where(d == dmin, iota, k), axis=1)
    return (iota == idx[:, None]).astype(jnp.float32)


def _dist(x, cb, cn):
    m = lax.dot_general(x, cb, (((1,), (1,)), ((), ())),
                        preferred_element_type=jnp.float32)
    s = jnp.sum(x * x, axis=1, keepdims=True)
    return (s - 2.0 * m) + cn


def _norms_body(cb_ref, cbs_ref, cn_ref, cbn_ref):
    cb0 = cb_ref[...]
    cn_ref[...] = jnp.sum(cb0 * cb0, axis=1)[None, :]
    for i in range(_NQ):
        cbi = cbs_ref[i]
        cbn_ref[i, :] = jnp.sum(cbi * cbi, axis=1)


def _norms(centroid, codebooks):
    k = centroid.shape[0]
    return pl.pallas_call(
        _norms_body,
        in_specs=[pl.BlockSpec(centroid.shape, lambda: (0, 0)),
                  pl.BlockSpec(codebooks.shape, lambda: (0, 0, 0))],
        out_specs=(pl.BlockSpec((1, k), lambda: (0, 0)),
                   pl.BlockSpec((_NQ, codebooks.shape[1]),
                                lambda: (0, 0))),
        out_shape=(jax.ShapeDtypeStruct((1, k), jnp.float32),
                   jax.ShapeDtypeStruct((_NQ, codebooks.shape[1]),
                                        jnp.float32)),
    )(centroid, codebooks)


def _linvq_idx_body(f_ref, cb_ref, cn_ref, out_ref):
    ft = f_ref[0].T
    d = _dist(ft, cb_ref[...], cn_ref[...])
    dmin = jnp.min(d, axis=1, keepdims=True)
    iota = lax.broadcasted_iota(jnp.int32, d.shape, 1)
    idx = jnp.min(jnp.where(d == dmin, iota, d.shape[1]), axis=1)
    tpad = out_ref.shape[2] - idx.shape[0]
    out_ref[...] = jnp.pad(idx, (0, tpad))[None, None, :]


def _mlp3(h, w1, b1, w2, b2, w3, b3, act, act_last):
    h = act(jnp.dot(h, w1, preferred_element_type=jnp.float32) + b1)
    h = act(jnp.dot(h, w2, preferred_element_type=jnp.float32) + b2)
    h = jnp.dot(h, w3, preferred_element_type=jnp.float32) + b3
    return act_last(h)


def _relu(x):
    return jax.nn.relu(x)


def _ident(x):
    return x


def _fused_body(lo_branch, f_ref, ld_ref,
                ew1, eb1, ew2, eb2, ew3, eb3,
                nw1, nb1, nw2, nb2, nw3, nb3,
                cbs_ref, cbn_ref, dw1, db1, dw2, db2, dw3, db3,
                out_ref):
    f = f_ref[0].T
    ld = ld_ref[0][:f.shape[0]]
    spk_raw = f - ld
    spk_enc = _mlp3(spk_raw, ew1[...], eb1[...], ew2[...], eb2[...],
                    ew3[...], eb3[...], _leaky, _ident)
    if lo_branch:
        ld_ref_enc = _mlp3(ld, ew1[...], eb1[...], ew2[...], eb2[...],
                           ew3[...], eb3[...], _leaky, _ident)
        nv = spk_enc - ld_ref_enc
    else:
        nv = _mlp3(ld, nw1[...], nb1[...], nw2[...], nb2[...],
                   nw3[...], nb3[...], _relu, _relu)
    sen = spk_enc / (nv + 1e-08)

    res = sen
    quant = jnp.zeros_like(sen)
    for i in range(_NQ):
        cbi = cbs_ref[i]
        oh = _argmin_onehot(_dist(res, cbi, cbn_ref[i][None, :]),
                            cbi.shape[0])
        q = jnp.dot(oh, cbi, preferred_element_type=jnp.float32)
        quant = quant + q
        res = res - q
    q_spk = sen + (quant - sen)
    den = q_spk * nv
    spk_dec = _mlp3(den, dw1[...], db1[...], dw2[...], db2[...],
                    dw3[...], db3[...], _leaky, _ident)
    out_ref[0] = (ld + spk_dec).T


def _full(shape):
    zeros = (0,) * len(shape)
    return pl.BlockSpec(shape, lambda i, z=zeros: z)


def _params():
    return pltpu.CompilerParams(dimension_semantics=("arbitrary",))


def _linvq_idx(feature, centroid, cn):
    b, c, t = feature.shape
    k = centroid.shape[0]
    return pl.pallas_call(
        _linvq_idx_body,
        grid=(b,),
        in_specs=[pl.BlockSpec((1, c, t), lambda i: (i, 0, 0)),
                  _full((k, c)), _full((1, k))],
        out_specs=pl.BlockSpec((1, 1, _TPAD), lambda i: (i, 0, 0)),
        out_shape=jax.ShapeDtypeStruct((b, 1, _TPAD), jnp.int32),
        compiler_params=_params(),
    )(feature, centroid, cn)


def _sc_gather(table, idx, rows_per_chunk):
    info = plsc.get_sparse_core_info()
    nw = info.num_cores * info.num_subcores
    b = idx.shape[0]
    d = table.shape[1]
    b_per_w = b // nw
    nchunks = b_per_w // rows_per_chunk
    mesh = plsc.VectorSubcoreMesh(core_axis_name="c", subcore_axis_name="s")

    def body(table_hbm, idx_hbm, out_hbm, idx_v, rows_v, sem):
        wid = lax.axis_index("s") * info.num_cores + lax.axis_index("c")
        base = wid * b_per_w
        for j in range(nchunks):
            off = base + j * rows_per_chunk
            pltpu.sync_copy(idx_hbm.at[pl.ds(off, rows_per_chunk)], idx_v)
            pltpu.async_copy(table_hbm.at[idx_v], rows_v, sem).wait()
            pltpu.sync_copy(rows_v, out_hbm.at[pl.ds(off, rows_per_chunk)])

    fn = pl.kernel(
        body,
        mesh=mesh,
        out_type=jax.ShapeDtypeStruct((b, d), jnp.float32),
        scratch_types=[
            pltpu.VMEM((rows_per_chunk,), jnp.int32),
            pltpu.VMEM((rows_per_chunk, d), jnp.float32),
            pltpu.SemaphoreType.DMA,
        ],
    )
    return fn(table, idx)


def _fused(lo_branch, feature, lin_dec,
           ew1, eb1, ew2, eb2, ew3, eb3,
           nw1, nb1, nw2, nb2, nw3, nb3,
           codebooks, cbn, dw1, db1, dw2, db2, dw3, db3):
    b, c, t = feature.shape
    wspecs = [_full(ew1.shape), _full(eb1.shape), _full(ew2.shape),
              _full(eb2.shape), _full(ew3.shape), _full(eb3.shape),
              _full(nw1.shape), _full(nb1.shape), _full(nw2.shape),
              _full(nb2.shape), _full(nw3.shape), _full(nb3.shape),
              _full(codebooks.shape), _full(cbn.shape),
              _full(dw1.shape), _full(db1.shape), _full(dw2.shape),
              _full(db2.shape), _full(dw3.shape), _full(db3.shape)]
    return pl.pallas_call(
        functools.partial(_fused_body, lo_branch),
        grid=(b,),
        in_specs=[pl.BlockSpec((1, c, t), lambda i: (i, 0, 0)),
                  pl.BlockSpec((1, _TPAD, c), lambda i: (i, 0, 0))] + wspecs,
        out_specs=pl.BlockSpec((1, c, t), lambda i: (i, 0, 0)),
        out_shape=jax.ShapeDtypeStruct((b, c, t), jnp.float32),
        compiler_params=_params(),
    )(feature, lin_dec, ew1, eb1, ew2, eb2, ew3, eb3,
      nw1, nb1, nw2, nb2, nw3, nb3,
      codebooks, cbn, dw1, db1, dw2, db2, dw3, db3)


def kernel(feature, centroid, enc_w1, enc_b1, enc_w2, enc_b2, enc_w3, enc_b3,
           dec_w1, dec_b1, dec_w2, dec_b2, dec_w3, dec_b3,
           nrm_w1, nrm_b1, nrm_w2, nrm_b2, nrm_w3, nrm_b3,
           codebooks, iteration):
    b, c, t = feature.shape
    n = b * t

    eb1, eb2, eb3 = enc_b1[None, :], enc_b2[None, :], enc_b3[None, :]
    nb1, nb2, nb3 = nrm_b1[None, :], nrm_b2[None, :], nrm_b3[None, :]
    db1, db2, db3 = dec_b1[None, :], dec_b2[None, :], dec_b3[None, :]

    del n
    cn, cbn = _norms(centroid, codebooks)
    idx = _linvq_idx(feature, centroid, cn).reshape(b * _TPAD)
    lin_dec = _sc_gather(centroid, idx, 64).reshape(b, _TPAD, c)

    args = (feature, lin_dec, enc_w1, eb1, enc_w2, eb2, enc_w3, eb3,
            nrm_w1, nb1, nrm_w2, nb2, nrm_w3, nb3,
            codebooks, cbn, dec_w1, db1, dec_w2, db2, dec_w3, db3)
    return lax.cond(
        iteration > _NORM_START,
        lambda *a: _fused(False, *a),
        lambda *a: _fused(True, *a),
        *args)


# speedup vs baseline: 1.0127x; 1.0127x over previous
import functools

import jax
import jax.numpy as jnp
from jax import lax
from jax.experimental import pallas as pl
from jax.experimental.pallas import tpu as pltpu
from jax.experimental.pallas import tpu_sc as plsc

_NORM_START = 5000
_NQ = 8
_TPAD = 512


def _leaky(x):
    return jnp.where(x >= 0, x, 0.01 * x)


def _argmin_onehot(d, k):
    dmin = jnp.min(d, axis=1, keepdims=True)
    iota = lax.broadcasted_iota(jnp.int32, d.shape, 1)
    idx = jnp.min(jnp.where(d == dmin, iota, k), axis=1)
    return (iota == idx[:, None]).astype(jnp.float32)


def _dist(x, cb, cn):
    m = lax.dot_general(x, cb, (((1,), (1,)), ((), ())),
                        preferred_element_type=jnp.float32)
    s = jnp.sum(x * x, axis=1, keepdims=True)
    return (s - 2.0 * m) + cn


def _linvq_idx_body(f_ref, cb_ref, out_ref, cn_ref):
    @pl.when(pl.program_id(0) == 0)
    def _():
        cb0 = cb_ref[...]
        cn_ref[...] = jnp.sum(cb0 * cb0, axis=1)[None, :]

    ft = f_ref[0].T
    d = _dist(ft, cb_ref[...], cn_ref[...])
    dmin = jnp.min(d, axis=1, keepdims=True)
    iota = lax.broadcasted_iota(jnp.int32, d.shape, 1)
    idx = jnp.min(jnp.where(d == dmin, iota, d.shape[1]), axis=1)
    tpad = out_ref.shape[2] - idx.shape[0]
    out_ref[...] = jnp.pad(idx, (0, tpad))[None, None, :]


def _mlp3(h, w1, b1, w2, b2, w3, b3, act, act_last):
    h = act(jnp.dot(h, w1, preferred_element_type=jnp.float32) + b1)
    h = act(jnp.dot(h, w2, preferred_element_type=jnp.float32) + b2)
    h = jnp.dot(h, w3, preferred_element_type=jnp.float32) + b3
    return act_last(h)


def _relu(x):
    return jax.nn.relu(x)


def _ident(x):
    return x


def _fused_body(lo_branch, f_ref, ld_ref,
                ew1, eb1, ew2, eb2, ew3, eb3,
                nw1, nb1, nw2, nb2, nw3, nb3,
                cbs_ref, dw1, db1, dw2, db2, dw3, db3,
                out_ref, cbn_ref):
    @pl.when(pl.program_id(0) == 0)
    def _():
        for i in range(_NQ):
            cbi = cbs_ref[i]
            cbn_ref[i, :] = jnp.sum(cbi * cbi, axis=1)

    f = f_ref[0].T
    ld = ld_ref[0][:f.shape[0]]
    spk_raw = f - ld
    spk_enc = _mlp3(spk_raw, ew1[...], eb1[...], ew2[...], eb2[...],
                    ew3[...], eb3[...], _leaky, _ident)
    if lo_branch:
        ld_ref_enc = _mlp3(ld, ew1[...], eb1[...], ew2[...], eb2[...],
                           ew3[...], eb3[...], _leaky, _ident)
        nv = spk_enc - ld_ref_enc
    else:
        nv = _mlp3(ld, nw1[...], nb1[...], nw2[...], nb2[...],
                   nw3[...], nb3[...], _relu, _relu)
    sen = spk_enc / (nv + 1e-08)

    res = sen
    quant = jnp.zeros_like(sen)
    for i in range(_NQ):
        cbi = cbs_ref[i]
        oh = _argmin_onehot(_dist(res, cbi, cbn_ref[i][None, :]),
                            cbi.shape[0])
        q = jnp.dot(oh, cbi, preferred_element_type=jnp.float32)
        quant = quant + q
        res = res - q
    q_spk = sen + (quant - sen)
    den = q_spk * nv
    spk_dec = _mlp3(den, dw1[...], db1[...], dw2[...], db2[...],
                    dw3[...], db3[...], _leaky, _ident)
    out_ref[0] = (ld + spk_dec).T


def _full(shape):
    zeros = (0,) * len(shape)
    return pl.BlockSpec(shape, lambda i, z=zeros: z)


def _params():
    return pltpu.CompilerParams(dimension_semantics=("arbitrary",))


def _linvq_idx(feature, centroid):
    b, c, t = feature.shape
    k = centroid.shape[0]
    return pl.pallas_call(
        _linvq_idx_body,
        grid=(b,),
        in_specs=[pl.BlockSpec((1, c, t), lambda i: (i, 0, 0)),
                  _full((k, c))],
        out_specs=pl.BlockSpec((1, 1, _TPAD), lambda i: (i, 0, 0)),
        out_shape=jax.ShapeDtypeStruct((b, 1, _TPAD), jnp.int32),
        scratch_shapes=[pltpu.VMEM((1, k), jnp.float32)],
        compiler_params=_params(),
    )(feature, centroid)


def _sc_gather(table, idx, rows_per_chunk):
    info = plsc.get_sparse_core_info()
    nw = info.num_cores * info.num_subcores
    b = idx.shape[0]
    d = table.shape[1]
    b_per_w = b // nw
    nchunks = b_per_w // rows_per_chunk
    mesh = plsc.VectorSubcoreMesh(core_axis_name="c", subcore_axis_name="s")

    def body(table_hbm, idx_hbm, out_hbm, idx_v, rows_v, sem):
        wid = lax.axis_index("s") * info.num_cores + lax.axis_index("c")
        base = wid * b_per_w
        for j in range(nchunks):
            off = base + j * rows_per_chunk
            pltpu.sync_copy(idx_hbm.at[pl.ds(off, rows_per_chunk)], idx_v)
            pltpu.async_copy(table_hbm.at[idx_v], rows_v, sem).wait()
            pltpu.sync_copy(rows_v, out_hbm.at[pl.ds(off, rows_per_chunk)])

    fn = pl.kernel(
        body,
        mesh=mesh,
        out_type=jax.ShapeDtypeStruct((b, d), jnp.float32),
        scratch_types=[
            pltpu.VMEM((rows_per_chunk,), jnp.int32),
            pltpu.VMEM((rows_per_chunk, d), jnp.float32),
            pltpu.SemaphoreType.DMA,
        ],
    )
    return fn(table, idx)


def _fused(lo_branch, feature, lin_dec,
           ew1, eb1, ew2, eb2, ew3, eb3,
           nw1, nb1, nw2, nb2, nw3, nb3,
           codebooks, dw1, db1, dw2, db2, dw3, db3):
    b, c, t = feature.shape
    wspecs = [_full(ew1.shape), _full(eb1.shape), _full(ew2.shape),
              _full(eb2.shape), _full(ew3.shape), _full(eb3.shape),
              _full(nw1.shape), _full(nb1.shape), _full(nw2.shape),
              _full(nb2.shape), _full(nw3.shape), _full(nb3.shape),
              _full(codebooks.shape),
              _full(dw1.shape), _full(db1.shape), _full(dw2.shape),
              _full(db2.shape), _full(dw3.shape), _full(db3.shape)]
    return pl.pallas_call(
        functools.partial(_fused_body, lo_branch),
        grid=(b,),
        in_specs=[pl.BlockSpec((1, c, t), lambda i: (i, 0, 0)),
                  pl.BlockSpec((1, _TPAD, c), lambda i: (i, 0, 0))] + wspecs,
        out_specs=pl.BlockSpec((1, c, t), lambda i: (i, 0, 0)),
        out_shape=jax.ShapeDtypeStruct((b, c, t), jnp.float32),
        scratch_shapes=[pltpu.VMEM((_NQ, codebooks.shape[1]), jnp.float32)],
        compiler_params=_params(),
    )(feature, lin_dec, ew1, eb1, ew2, eb2, ew3, eb3,
      nw1, nb1, nw2, nb2, nw3, nb3,
      codebooks, dw1, db1, dw2, db2, dw3, db3)


def kernel(feature, centroid, enc_w1, enc_b1, enc_w2, enc_b2, enc_w3, enc_b3,
           dec_w1, dec_b1, dec_w2, dec_b2, dec_w3, dec_b3,
           nrm_w1, nrm_b1, nrm_w2, nrm_b2, nrm_w3, nrm_b3,
           codebooks, iteration):
    b, c, t = feature.shape
    n = b * t

    eb1, eb2, eb3 = enc_b1[None, :], enc_b2[None, :], enc_b3[None, :]
    nb1, nb2, nb3 = nrm_b1[None, :], nrm_b2[None, :], nrm_b3[None, :]
    db1, db2, db3 = dec_b1[None, :], dec_b2[None, :], dec_b3[None, :]

    del n
    idx = _linvq_idx(feature, centroid).reshape(b * _TPAD)
    lin_dec = _sc_gather(centroid, idx, 64).reshape(b, _TPAD, c)

    args = (feature, lin_dec, enc_w1, eb1, enc_w2, eb2, enc_w3, eb3,
            nrm_w1, nb1, nrm_w2, nb2, nrm_w3, nb3,
            codebooks, dec_w1, db1, dec_w2, db2, dec_w3, db3)
    return lax.cond(
        iteration > _NORM_START,
        lambda *a: _fused(False, *a),
        lambda *a: _fused(True, *a),
        *args)
